# Initial kernel scaffold; baseline (speedup 1.0000x reference)
import jax
import jax.numpy as jnp
from jax.experimental import pallas as pl


def kernel(x, edge_index, W0, b0, W1, b1, W2, b2, g0, be0, g1, be1):
    raise NotImplementedError("write your pallas kernel here")


# speedup vs baseline: 13.0148x; 13.0148x over previous
import functools

import jax
import jax.numpy as jnp
from jax import lax
from jax.experimental import pallas as pl
from jax.experimental.pallas import tpu as pltpu
from jax.experimental.pallas import tpu_sc as plsc

N = 10000
E = 320000
D = 128
BN_EPS = 1e-5

NC = 2
NS = 16
NW = NC * NS
EPW = E // NW
CHUNK = 128
NFULL = EPW // CHUNK
TAIL = EPW - NFULL * CHUNK
DEG_W = 16
NPAD = 10240
DROWS_PER_SUB = NPAD // NS

_mesh = plsc.VectorSubcoreMesh(core_axis_name="c", subcore_axis_name="s")


@functools.partial(
    pl.kernel,
    out_type=jax.ShapeDtypeStruct((NC, NPAD, DEG_W), jnp.float32),
    mesh=_mesh,
    scratch_types=[
        pltpu.VMEM((1, CHUNK), jnp.int32),
        pltpu.VMEM((1, TAIL), jnp.int32),
        pltpu.VMEM((CHUNK, DEG_W), jnp.float32),
        pltpu.VMEM_SHARED((NPAD, DEG_W), jnp.float32),
    ],
    compiler_params=pltpu.CompilerParams(use_tc_tiling_on_sc=False),
)
def _deg_kernel(dst_hbm, ones_hbm, zeros_hbm, out_hbm,
                dst_v, dst_t, ones_v, acc):
    c = lax.axis_index("c")
    s = lax.axis_index("s")
    wid = c * NS + s
    base = wid * EPW

    pltpu.sync_copy(ones_hbm, ones_v)
    r0 = s * DROWS_PER_SUB
    pltpu.sync_copy(zeros_hbm.at[pl.ds(r0, DROWS_PER_SUB)],
                    acc.at[pl.ds(r0, DROWS_PER_SUB)])
    plsc.subcore_barrier()

    @pl.loop(0, NFULL)
    def _edge_loop(ci):
        off = base + ci * CHUNK
        pltpu.sync_copy(dst_hbm.at[pl.ds(off, CHUNK)], dst_v.at[0])
        pltpu.sync_copy(ones_v, acc.at[dst_v.at[0]], add=True)

    off = base + NFULL * CHUNK
    pltpu.sync_copy(dst_hbm.at[pl.ds(off, TAIL)], dst_t.at[0])
    pltpu.sync_copy(ones_v.at[pl.ds(0, TAIL)], acc.at[dst_t.at[0]], add=True)

    plsc.subcore_barrier()
    pltpu.sync_copy(acc.at[pl.ds(r0, DROWS_PER_SUB)],
                    out_hbm.at[c, pl.ds(r0, DROWS_PER_SUB)])


@functools.partial(
    pl.kernel,
    out_type=jax.ShapeDtypeStruct((NC, NPAD, D), jnp.float32),
    mesh=_mesh,
    scratch_types=[
        pltpu.VMEM((CHUNK,), jnp.int32),
        pltpu.VMEM((1, CHUNK), jnp.int32),
        pltpu.VMEM((TAIL,), jnp.int32),
        pltpu.VMEM((1, TAIL), jnp.int32),
        pltpu.VMEM((CHUNK, D), jnp.float32),
        pltpu.VMEM((TAIL, D), jnp.float32),
        pltpu.VMEM_SHARED((NPAD, D), jnp.float32),
    ],
)
def _agg_kernel(src_hbm, dst_hbm, hp_hbm, zeros_hbm, out_hbm,
                src_v, dst_v, src_t, dst_t, rows_v, rows_t, acc):
    c = lax.axis_index("c")
    s = lax.axis_index("s")
    wid = c * NS + s
    base = wid * EPW

    r0 = s * DROWS_PER_SUB
    pltpu.sync_copy(zeros_hbm.at[pl.ds(r0, DROWS_PER_SUB)],
                    acc.at[pl.ds(r0, DROWS_PER_SUB)])
    plsc.subcore_barrier()

    @pl.loop(0, NFULL)
    def _edge_loop(ci):
        off = base + ci * CHUNK
        pltpu.sync_copy(src_hbm.at[pl.ds(off, CHUNK)], src_v)
        pltpu.sync_copy(dst_hbm.at[pl.ds(off, CHUNK)], dst_v.at[0])
        pltpu.sync_copy(hp_hbm.at[src_v], rows_v)
        pltpu.sync_copy(rows_v, acc.at[dst_v.at[0]], add=True)

    off = base + NFULL * CHUNK
    pltpu.sync_copy(src_hbm.at[pl.ds(off, TAIL)], src_t)
    pltpu.sync_copy(dst_hbm.at[pl.ds(off, TAIL)], dst_t.at[0])
    pltpu.sync_copy(hp_hbm.at[src_t], rows_t)
    pltpu.sync_copy(rows_t, acc.at[dst_t.at[0]], add=True)

    plsc.subcore_barrier()
    pltpu.sync_copy(acc.at[pl.ds(r0, DROWS_PER_SUB)],
                    out_hbm.at[c, pl.ds(r0, DROWS_PER_SUB)])


_RB = 2000


def _mm_body(x_ref, w_ref, o_ref):
    o_ref[...] = jnp.dot(x_ref[...], w_ref[...],
                         preferred_element_type=jnp.float32)


def _matmul(x, w):
    return pl.pallas_call(
        _mm_body,
        out_shape=jax.ShapeDtypeStruct((N, D), jnp.float32),
        grid=(N // _RB,),
        in_specs=[
            pl.BlockSpec((_RB, D), lambda i: (i, 0)),
            pl.BlockSpec((D, D), lambda i: (0, 0)),
        ],
        out_specs=pl.BlockSpec((_RB, D), lambda i: (i, 0)),
    )(x, w)


def _prep_body(deg_ref, h0_ref, dis_ref, hp_ref):
    dblk = deg_ref[...]
    total = dblk[0, :, 0:1] + dblk[1, :, 0:1] + 1.0
    dis = lax.rsqrt(total)
    dis_ref[...] = jnp.broadcast_to(dis, (_RB, D))
    hp_ref[...] = h0_ref[...] * dis


def _prep(deg, h0):
    return pl.pallas_call(
        _prep_body,
        out_shape=(
            jax.ShapeDtypeStruct((N, D), jnp.float32),
            jax.ShapeDtypeStruct((N, D), jnp.float32),
        ),
        grid=(N // _RB,),
        in_specs=[
            pl.BlockSpec((NC, _RB, DEG_W), lambda i: (0, i, 0)),
            pl.BlockSpec((_RB, D), lambda i: (i, 0)),
        ],
        out_specs=(
            pl.BlockSpec((_RB, D), lambda i: (i, 0)),
            pl.BlockSpec((_RB, D), lambda i: (i, 0)),
        ),
    )(deg, h0)


def _mid_body(agg_ref, hp_ref, res_ref, dis_ref, b_ref, g_ref, be_ref, w_ref,
              y_ref, hpn_ref):
    inv = 1.0 / (1.0 + BN_EPS) ** 0.5
    ablk = agg_ref[...]
    dis = dis_ref[...]
    z = dis * (ablk[0] + ablk[1] + hp_ref[...]) + b_ref[...]
    z = z * (g_ref[...] * inv) + be_ref[...]
    y = jnp.maximum(z, 0.0) + res_ref[...]
    y_ref[...] = y
    hpn_ref[...] = dis * jnp.dot(y, w_ref[...],
                                 preferred_element_type=jnp.float32)


def _mid(agg, hp, res, dis, b, g, be, w):
    return pl.pallas_call(
        _mid_body,
        out_shape=(
            jax.ShapeDtypeStruct((N, D), jnp.float32),
            jax.ShapeDtypeStruct((N, D), jnp.float32),
        ),
        grid=(N // _RB,),
        in_specs=[
            pl.BlockSpec((NC, _RB, D), lambda i: (0, i, 0)),
            pl.BlockSpec((_RB, D), lambda i: (i, 0)),
            pl.BlockSpec((_RB, D), lambda i: (i, 0)),
            pl.BlockSpec((_RB, D), lambda i: (i, 0)),
            pl.BlockSpec((1, D), lambda i: (0, 0)),
            pl.BlockSpec((1, D), lambda i: (0, 0)),
            pl.BlockSpec((1, D), lambda i: (0, 0)),
            pl.BlockSpec((D, D), lambda i: (0, 0)),
        ],
        out_specs=(
            pl.BlockSpec((_RB, D), lambda i: (i, 0)),
            pl.BlockSpec((_RB, D), lambda i: (i, 0)),
        ),
    )(agg, hp, res, dis, b, g, be, w)


def _final_body(agg_ref, hp_ref, dis_ref, b_ref, o_ref):
    ablk = agg_ref[...]
    o_ref[...] = dis_ref[...] * (ablk[0] + ablk[1] + hp_ref[...]) + b_ref[...]


def _final(agg, hp, dis, b):
    return pl.pallas_call(
        _final_body,
        out_shape=jax.ShapeDtypeStruct((N, D), jnp.float32),
        grid=(N // _RB,),
        in_specs=[
            pl.BlockSpec((NC, _RB, D), lambda i: (0, i, 0)),
            pl.BlockSpec((_RB, D), lambda i: (i, 0)),
            pl.BlockSpec((_RB, D), lambda i: (i, 0)),
            pl.BlockSpec((1, D), lambda i: (0, 0)),
        ],
        out_specs=pl.BlockSpec((_RB, D), lambda i: (i, 0)),
    )(agg, hp, dis, b)


def kernel(x, edge_index, W0, b0, W1, b1, W2, b2, g0, be0, g1, be1):
    edge_index = edge_index.astype(jnp.int32)
    src = edge_index[0]
    dst = edge_index[1]
    zeros_nd = jnp.zeros((NPAD, D), jnp.float32)
    zeros_deg = jnp.zeros((NPAD, DEG_W), jnp.float32)
    ones_rows = jnp.ones((CHUNK, DEG_W), jnp.float32)
    b0r = b0.reshape(1, D)
    b1r = b1.reshape(1, D)
    b2r = b2.reshape(1, D)
    g0r = g0.reshape(1, D)
    g1r = g1.reshape(1, D)
    be0r = be0.reshape(1, D)
    be1r = be1.reshape(1, D)

    deg = _deg_kernel(dst, ones_rows, zeros_deg)
    h0 = _matmul(x, W0)
    dis, hp0 = _prep(deg, h0)

    agg0 = _agg_kernel(src, dst, hp0, zeros_nd)
    y1, hp1 = _mid(agg0, hp0, x, dis, b0r, g0r, be0r, W1)

    agg1 = _agg_kernel(src, dst, hp1, zeros_nd)
    y2, hp2 = _mid(agg1, hp1, y1, dis, b1r, g1r, be1r, W2)

    agg2 = _agg_kernel(src, dst, hp2, zeros_nd)
    return _final(agg2, hp2, dis, b2r)


# speedup vs baseline: 21.4808x; 1.6505x over previous
import functools

import jax
import jax.numpy as jnp
from jax import lax
from jax.experimental import pallas as pl
from jax.experimental.pallas import tpu as pltpu
from jax.experimental.pallas import tpu_sc as plsc

N = 10000
E = 320000
D = 128
BN_EPS = 1e-5

NC = 2
NS = 16
NW = NC * NS
EPW = E // NW
CHUNK = 128
NFULL = EPW // CHUNK
TAIL = EPW - NFULL * CHUNK
DEG_W = 16
NPAD = 10240
DROWS_PER_SUB = NPAD // NS

_mesh = plsc.VectorSubcoreMesh(core_axis_name="c", subcore_axis_name="s")


@functools.partial(
    pl.kernel,
    out_type=jax.ShapeDtypeStruct((NC, NPAD, DEG_W), jnp.float32),
    mesh=_mesh,
    scratch_types=[
        pltpu.VMEM((1, CHUNK), jnp.int32),
        pltpu.VMEM((1, TAIL), jnp.int32),
        pltpu.VMEM((CHUNK, DEG_W), jnp.float32),
        pltpu.VMEM_SHARED((NPAD, DEG_W), jnp.float32),
    ],
    compiler_params=pltpu.CompilerParams(use_tc_tiling_on_sc=False),
)
def _deg_kernel(dst_hbm, ones_hbm, zeros_hbm, out_hbm,
                dst_v, dst_t, ones_v, acc):
    c = lax.axis_index("c")
    s = lax.axis_index("s")
    wid = c * NS + s
    base = wid * EPW

    pltpu.sync_copy(ones_hbm, ones_v)
    r0 = s * DROWS_PER_SUB
    pltpu.sync_copy(zeros_hbm.at[pl.ds(r0, DROWS_PER_SUB)],
                    acc.at[pl.ds(r0, DROWS_PER_SUB)])
    plsc.subcore_barrier()

    @pl.loop(0, NFULL)
    def _edge_loop(ci):
        off = base + ci * CHUNK
        pltpu.sync_copy(dst_hbm.at[pl.ds(off, CHUNK)], dst_v.at[0])
        pltpu.sync_copy(ones_v, acc.at[dst_v.at[0]], add=True)

    off = base + NFULL * CHUNK
    pltpu.sync_copy(dst_hbm.at[pl.ds(off, TAIL)], dst_t.at[0])
    pltpu.sync_copy(ones_v.at[pl.ds(0, TAIL)], acc.at[dst_t.at[0]], add=True)

    plsc.subcore_barrier()
    pltpu.sync_copy(acc.at[pl.ds(r0, DROWS_PER_SUB)],
                    out_hbm.at[c, pl.ds(r0, DROWS_PER_SUB)])


NCK = 78
XN = (E - NW * NCK * CHUNK) // CHUNK
RPS = N // NS


def _unpack_chunk(pk_ref, ci, src_c, dst_c, b):
    for k in range(CHUNK // 16):
        p = pk_ref[ci, pl.ds(k * 16, 16)]
        src_c[b, pl.ds(k * 16, 16)] = p & 0xFFFF
        dst_c[b, pl.ds(k * 16, 16)] = lax.shift_right_logical(p, 16)


@functools.partial(
    pl.kernel,
    out_type=jax.ShapeDtypeStruct((NC, N, D), jnp.float32),
    mesh=_mesh,
    scratch_types=[
        pltpu.VMEM((NCK, CHUNK), jnp.int32),
        pltpu.VMEM((1, CHUNK), jnp.int32),
        pltpu.VMEM((2, CHUNK), jnp.int32),
        pltpu.VMEM((2, CHUNK), jnp.int32),
        pltpu.VMEM((CHUNK, D), jnp.float32),
        pltpu.VMEM((CHUNK, D), jnp.float32),
        pltpu.VMEM_SHARED((N, D), jnp.float32),
        pltpu.SemaphoreType.DMA,
        pltpu.SemaphoreType.DMA,
    ],
    compiler_params=pltpu.CompilerParams(use_tc_tiling_on_sc=False),
)
def _agg_kernel(pkR_hbm, pkX_hbm, hp_hbm, zeros_hbm,
                out_hbm, pk_v, pkx_v, src_c, dst_c, rows0, rows1, acc, g0, g1):
    c = lax.axis_index("c")
    s = lax.axis_index("s")
    wid = c * NS + s

    r0 = s * RPS
    pltpu.sync_copy(zeros_hbm.at[pl.ds(r0, RPS)], acc.at[pl.ds(r0, RPS)])
    pltpu.sync_copy(pkR_hbm.at[wid], pk_v)

    _unpack_chunk(pk_v, 0, src_c, dst_c, 0)
    pltpu.async_copy(hp_hbm.at[src_c.at[0]], rows0, g0)
    plsc.subcore_barrier()

    @pl.loop(0, NCK // 2)
    def _edge_loop(j):
        i0 = 2 * j
        _unpack_chunk(pk_v, i0 + 1, src_c, dst_c, 1)
        pltpu.make_async_copy(hp_hbm.at[src_c.at[0]], rows0, g0).wait()
        pltpu.async_copy(hp_hbm.at[src_c.at[1]], rows1, g1)
        pltpu.sync_copy(rows0, acc.at[dst_c.at[0]], add=True)

        @pl.when(j < NCK // 2 - 1)
        def _():
            _unpack_chunk(pk_v, i0 + 2, src_c, dst_c, 0)

        pltpu.make_async_copy(hp_hbm.at[src_c.at[1]], rows1, g1).wait()

        @pl.when(j < NCK // 2 - 1)
        def _():
            pltpu.async_copy(hp_hbm.at[src_c.at[0]], rows0, g0)

        pltpu.sync_copy(rows1, acc.at[dst_c.at[1]], add=True)

    @pl.when(wid < XN)
    def _():
        pltpu.sync_copy(pkX_hbm.at[wid], pkx_v.at[0])
        _unpack_chunk(pkx_v, 0, src_c, dst_c, 0)
        pltpu.sync_copy(hp_hbm.at[src_c.at[0]], rows0)
        pltpu.sync_copy(rows0, acc.at[dst_c.at[0]], add=True)

    plsc.subcore_barrier()
    pltpu.sync_copy(acc.at[pl.ds(r0, RPS)], out_hbm.at[c, pl.ds(r0, RPS)])


_RB = 2000


def _mm_body(x_ref, w_ref, o_ref):
    o_ref[...] = jnp.dot(x_ref[...], w_ref[...],
                         preferred_element_type=jnp.float32)


def _matmul(x, w):
    return pl.pallas_call(
        _mm_body,
        out_shape=jax.ShapeDtypeStruct((N, D), jnp.float32),
        grid=(N // _RB,),
        in_specs=[
            pl.BlockSpec((_RB, D), lambda i: (i, 0)),
            pl.BlockSpec((D, D), lambda i: (0, 0)),
        ],
        out_specs=pl.BlockSpec((_RB, D), lambda i: (i, 0)),
    )(x, w)


def _prep_body(deg_ref, h0_ref, dis_ref, hp_ref):
    dblk = deg_ref[...]
    total = dblk[0, :, 0:1] + dblk[1, :, 0:1] + 1.0
    dis = lax.rsqrt(total)
    dis_ref[...] = jnp.broadcast_to(dis, (_RB, D))
    hp_ref[...] = h0_ref[...] * dis


def _prep(deg, h0):
    return pl.pallas_call(
        _prep_body,
        out_shape=(
            jax.ShapeDtypeStruct((N, D), jnp.float32),
            jax.ShapeDtypeStruct((N, D), jnp.float32),
        ),
        grid=(N // _RB,),
        in_specs=[
            pl.BlockSpec((NC, _RB, DEG_W), lambda i: (0, i, 0)),
            pl.BlockSpec((_RB, D), lambda i: (i, 0)),
        ],
        out_specs=(
            pl.BlockSpec((_RB, D), lambda i: (i, 0)),
            pl.BlockSpec((_RB, D), lambda i: (i, 0)),
        ),
    )(deg, h0)


def _mid_body(agg_ref, hp_ref, res_ref, dis_ref, b_ref, g_ref, be_ref, w_ref,
              y_ref, hpn_ref):
    inv = 1.0 / (1.0 + BN_EPS) ** 0.5
    ablk = agg_ref[...]
    dis = dis_ref[...]
    z = dis * (ablk[0] + ablk[1] + hp_ref[...]) + b_ref[...]
    z = z * (g_ref[...] * inv) + be_ref[...]
    y = jnp.maximum(z, 0.0) + res_ref[...]
    y_ref[...] = y
    hpn_ref[...] = dis * jnp.dot(y, w_ref[...],
                                 preferred_element_type=jnp.float32)


def _mid(agg, hp, res, dis, b, g, be, w):
    return pl.pallas_call(
        _mid_body,
        out_shape=(
            jax.ShapeDtypeStruct((N, D), jnp.float32),
            jax.ShapeDtypeStruct((N, D), jnp.float32),
        ),
        grid=(N // _RB,),
        in_specs=[
            pl.BlockSpec((NC, _RB, D), lambda i: (0, i, 0)),
            pl.BlockSpec((_RB, D), lambda i: (i, 0)),
            pl.BlockSpec((_RB, D), lambda i: (i, 0)),
            pl.BlockSpec((_RB, D), lambda i: (i, 0)),
            pl.BlockSpec((1, D), lambda i: (0, 0)),
            pl.BlockSpec((1, D), lambda i: (0, 0)),
            pl.BlockSpec((1, D), lambda i: (0, 0)),
            pl.BlockSpec((D, D), lambda i: (0, 0)),
        ],
        out_specs=(
            pl.BlockSpec((_RB, D), lambda i: (i, 0)),
            pl.BlockSpec((_RB, D), lambda i: (i, 0)),
        ),
    )(agg, hp, res, dis, b, g, be, w)


def _final_body(agg_ref, hp_ref, dis_ref, b_ref, o_ref):
    ablk = agg_ref[...]
    o_ref[...] = dis_ref[...] * (ablk[0] + ablk[1] + hp_ref[...]) + b_ref[...]


def _final(agg, hp, dis, b):
    return pl.pallas_call(
        _final_body,
        out_shape=jax.ShapeDtypeStruct((N, D), jnp.float32),
        grid=(N // _RB,),
        in_specs=[
            pl.BlockSpec((NC, _RB, D), lambda i: (0, i, 0)),
            pl.BlockSpec((_RB, D), lambda i: (i, 0)),
            pl.BlockSpec((_RB, D), lambda i: (i, 0)),
            pl.BlockSpec((1, D), lambda i: (0, 0)),
        ],
        out_specs=pl.BlockSpec((_RB, D), lambda i: (i, 0)),
    )(agg, hp, dis, b)


def kernel(x, edge_index, W0, b0, W1, b1, W2, b2, g0, be0, g1, be1):
    edge_index = edge_index.astype(jnp.int32)
    src = edge_index[0]
    dst = edge_index[1]
    nmain = NW * NCK * CHUNK
    packed = jnp.bitwise_or(jnp.left_shift(dst, 16), src)
    pkR = packed[:nmain].reshape(NW, NCK, CHUNK)
    pkX = packed[nmain:].reshape(XN, CHUNK)
    zeros_nd = jnp.zeros((N, D), jnp.float32)
    zeros_deg = jnp.zeros((NPAD, DEG_W), jnp.float32)
    ones_rows = jnp.ones((CHUNK, DEG_W), jnp.float32)
    b0r = b0.reshape(1, D)
    b1r = b1.reshape(1, D)
    b2r = b2.reshape(1, D)
    g0r = g0.reshape(1, D)
    g1r = g1.reshape(1, D)
    be0r = be0.reshape(1, D)
    be1r = be1.reshape(1, D)

    deg = _deg_kernel(dst, ones_rows, zeros_deg)
    h0 = _matmul(x, W0)
    dis, hp0 = _prep(deg, h0)

    agg0 = _agg_kernel(pkR, pkX, hp0, zeros_nd)
    y1, hp1 = _mid(agg0, hp0, x, dis, b0r, g0r, be0r, W1)

    agg1 = _agg_kernel(pkR, pkX, hp1, zeros_nd)
    y2, hp2 = _mid(agg1, hp1, y1, dis, b1r, g1r, be1r, W2)

    agg2 = _agg_kernel(pkR, pkX, hp2, zeros_nd)
    return _final(agg2, hp2, dis, b2r)
